# baseline (speedup 1.0000x reference)
import functools
import math

import jax
import jax.numpy as jnp
from jax import lax
from jax.experimental import pallas as pl
from jax.experimental.pallas import tpu as pltpu
from jax.experimental.pallas import tpu_sc as plsc

_D = 512
_SCALE = math.sqrt(_D)
_NC, _NS = 2, 16
_NW = _NC * _NS
_CHUNK = 32
_NBUF = 3
_LANES = 16


def _make_scaled_gather(bsz, seq, d):
    n = bsz * seq
    per_w = n // _NW
    w_per_b = seq // per_w
    n_chunks = per_w // _CHUNK
    cw = _CHUNK * d
    mesh = plsc.VectorSubcoreMesh(
        core_axis_name="c", subcore_axis_name="s",
        num_cores=_NC, num_subcores=_NS)

    @functools.partial(
        pl.kernel,
        out_type=jax.ShapeDtypeStruct((n * d,), jnp.float32),
        mesh=mesh,
        scratch_types=[
            pltpu.VMEM((per_w,), jnp.int32),
            *[pltpu.VMEM((_CHUNK, d), jnp.float32) for _ in range(_NBUF)],
            *[pltpu.VMEM((cw,), jnp.float32) for _ in range(_NBUF)],
            *[pltpu.SemaphoreType.DMA for _ in range(2 * _NBUF)],
        ],
    )
    def emb(x_hbm, lut_hbm, out_hbm, idx_v, *rest):
        rows = rest[:_NBUF]
        flats = rest[_NBUF:2 * _NBUF]
        in_sems = rest[2 * _NBUF:3 * _NBUF]
        out_sems = rest[3 * _NBUF:]
        wid = lax.axis_index("s") * _NC + lax.axis_index("c")
        b_i = wid // w_per_b
        col = (wid % w_per_b) * per_w
        pltpu.sync_copy(x_hbm.at[b_i, pl.ds(col, per_w)], idx_v)
        base = wid * per_w * d

        def start_gather(c, b):
            pltpu.async_copy(
                lut_hbm.at[idx_v.at[pl.ds(c * _CHUNK, _CHUNK)]],
                rows[b], in_sems[b])

        def wait_gather(b):
            pltpu.make_async_copy(
                lut_hbm.at[idx_v.at[pl.ds(0, _CHUNK)]],
                rows[b], in_sems[b]).wait()

        def start_store(c, b):
            pltpu.async_copy(
                flats[b], out_hbm.at[pl.ds(base + c * cw, cw)], out_sems[b])

        def wait_store(b):
            pltpu.make_async_copy(
                flats[b], out_hbm.at[pl.ds(base, cw)], out_sems[b]).wait()

        def scale(b):
            def row_body(r, acc):
                off = r * d
                for j in range(d // _LANES):
                    flats[b][pl.ds(off + j * _LANES, _LANES)] = (
                        rows[b][r, pl.ds(j * _LANES, _LANES)] * _SCALE)
                return acc
            lax.fori_loop(0, _CHUNK, row_body, 0)

        def step(c, b, *, prefetch=True, drain=True):
            if prefetch:
                start_gather(c + 2, (b + 2) % _NBUF)
            wait_gather(b)
            if drain:
                wait_store(b)
            scale(b)
            start_store(c, b)

        start_gather(0, 0)
        start_gather(1, 1)
        step(0, 0, drain=False)
        step(1, 1, drain=False)
        step(2, 2, drain=False)

        def group(g, acc):
            c0 = g * _NBUF
            step(c0, 0)
            step(c0 + 1, 1)
            step(c0 + 2, 2)
            return acc
        lax.fori_loop(1, n_chunks // _NBUF, group, 0)

        step(n_chunks - 2, (n_chunks - 2) % _NBUF, prefetch=False)
        step(n_chunks - 1, (n_chunks - 1) % _NBUF, prefetch=False)
        for b in range(_NBUF):
            wait_store(b)

    return emb


def kernel(x, lut):
    bsz, seq = x.shape
    out = _make_scaled_gather(bsz, seq, lut.shape[1])(
        x.astype(jnp.int32), lut)
    return out.reshape(bsz, seq, lut.shape[1])


# speedup vs baseline: 3.3012x; 3.3012x over previous
import functools
import math

import jax
import jax.numpy as jnp
from jax import lax
from jax.experimental import pallas as pl
from jax.experimental.pallas import tpu as pltpu
from jax.experimental.pallas import tpu_sc as plsc

_D = 512
_SCALE = math.sqrt(_D)
_NC, _NS = 2, 16
_NW = _NC * _NS
_CHUNK = 64
_NBUF = 3
_LANES = 16


def _make_scaled_gather(bsz, seq, d):
    n = bsz * seq
    per_w = n // _NW
    w_per_b = seq // per_w
    n_chunks = per_w // _CHUNK
    mesh = plsc.VectorSubcoreMesh(
        core_axis_name="c", subcore_axis_name="s",
        num_cores=_NC, num_subcores=_NS)

    @functools.partial(
        pl.kernel,
        out_type=jax.ShapeDtypeStruct((bsz, seq, d), jnp.float32),
        mesh=mesh,
        scratch_types=[
            pltpu.VMEM((per_w,), jnp.int32),
            *[pltpu.VMEM((_CHUNK, d), jnp.float32) for _ in range(_NBUF)],
            *[pltpu.SemaphoreType.DMA for _ in range(2 * _NBUF)],
        ],
    )
    def emb(x_hbm, lut_hbm, out_hbm, idx_v, *rest):
        rows = rest[:_NBUF]
        in_sems = rest[_NBUF:2 * _NBUF]
        out_sems = rest[2 * _NBUF:]
        wid = lax.axis_index("s") * _NC + lax.axis_index("c")
        b_i = wid // w_per_b
        col = (wid % w_per_b) * per_w
        pltpu.sync_copy(x_hbm.at[b_i, pl.ds(col, per_w)], idx_v)

        def start_gather(c):
            b = c % _NBUF
            return pltpu.async_copy(
                lut_hbm.at[idx_v.at[pl.ds(c * _CHUNK, _CHUNK)]],
                rows[b], in_sems[b])

        gathers = {0: start_gather(0)}
        stores = {}
        for c in range(n_chunks):
            b = c % _NBUF
            nxt = c + 1
            if nxt < n_chunks:
                if nxt - _NBUF in stores:
                    for s in stores.pop(nxt - _NBUF):
                        s.wait()
                gathers[nxt] = start_gather(nxt)
            gathers.pop(c).wait()

            def row_body(r, acc, _b=b):
                for j in range(d // _LANES):
                    sl = (r, pl.ds(j * _LANES, _LANES))
                    rows[_b][sl] = rows[_b][sl] * _SCALE
                return acc
            half = _CHUNK // 2
            lax.fori_loop(0, half, row_body, 0)
            s1 = pltpu.async_copy(
                rows[b].at[pl.ds(0, half)],
                out_hbm.at[b_i, pl.ds(col + c * _CHUNK, half)],
                out_sems[b])
            lax.fori_loop(half, _CHUNK, row_body, 0)
            s2 = pltpu.async_copy(
                rows[b].at[pl.ds(half, half)],
                out_hbm.at[b_i, pl.ds(col + c * _CHUNK + half, half)],
                out_sems[b])
            stores[c] = (s1, s2)
        for c in sorted(stores):
            for s in stores.pop(c):
                s.wait()

    return emb


def kernel(x, lut):
    bsz, seq = x.shape
    return _make_scaled_gather(bsz, seq, lut.shape[1])(
        x.astype(jnp.int32), lut)


# speedup vs baseline: 3.4251x; 1.0375x over previous
import functools
import math

import jax
import jax.numpy as jnp
from jax import lax
from jax.experimental import pallas as pl
from jax.experimental.pallas import tpu as pltpu
from jax.experimental.pallas import tpu_sc as plsc

_D = 512
_SCALE = math.sqrt(_D)
_NC, _NS = 2, 16
_NW = _NC * _NS
_CHUNK = 64
_NBUF = 3
_LANES = 16


def _make_scaled_gather(bsz, seq, d):
    n = bsz * seq
    per_w = n // _NW
    w_per_b = seq // per_w
    n_chunks = per_w // _CHUNK
    mesh = plsc.VectorSubcoreMesh(
        core_axis_name="c", subcore_axis_name="s",
        num_cores=_NC, num_subcores=_NS)

    @functools.partial(
        pl.kernel,
        out_type=jax.ShapeDtypeStruct((bsz, seq, d), jnp.float32),
        mesh=mesh,
        scratch_types=[
            pltpu.VMEM((per_w,), jnp.int32),
            *[pltpu.VMEM((_CHUNK, d), jnp.float32) for _ in range(_NBUF)],
            *[pltpu.SemaphoreType.DMA for _ in range(2 * _NBUF)],
        ],
    )
    def emb(x_hbm, lut_hbm, out_hbm, idx_v, *rest):
        rows = rest[:_NBUF]
        in_sems = rest[_NBUF:2 * _NBUF]
        out_sems = rest[2 * _NBUF:]
        wid = lax.axis_index("s") * _NC + lax.axis_index("c")
        b_i = wid // w_per_b
        col = (wid % w_per_b) * per_w
        pltpu.sync_copy(x_hbm.at[b_i, pl.ds(col, per_w)], idx_v)

        def start_gather(c):
            b = c % _NBUF
            return pltpu.async_copy(
                lut_hbm.at[idx_v.at[pl.ds(c * _CHUNK, _CHUNK)]],
                rows[b], in_sems[b])

        gathers = {0: start_gather(0)}
        stores = {}
        for c in range(n_chunks):
            b = c % _NBUF
            nxt = c + 1
            if nxt < n_chunks:
                if nxt - _NBUF in stores:
                    stores.pop(nxt - _NBUF).wait()
                gathers[nxt] = start_gather(nxt)
            gathers.pop(c).wait()

            def row_body(r, acc, _b=b):
                for j in range(d // _LANES):
                    sl = (r, pl.ds(j * _LANES, _LANES))
                    rows[_b][sl] = rows[_b][sl] * _SCALE
                return acc
            lax.fori_loop(0, _CHUNK, row_body, 0)

            stores[c] = pltpu.async_copy(
                rows[b],
                out_hbm.at[b_i, pl.ds(col + c * _CHUNK, _CHUNK)],
                out_sems[b])
        for c in sorted(stores):
            stores.pop(c).wait()

    return emb


def kernel(x, lut):
    bsz, seq = x.shape
    return _make_scaled_gather(bsz, seq, lut.shape[1])(
        x.astype(jnp.int32), lut)


# speedup vs baseline: 3.4382x; 1.0038x over previous
import functools
import math

import jax
import jax.numpy as jnp
from jax import lax
from jax.experimental import pallas as pl
from jax.experimental.pallas import tpu as pltpu
from jax.experimental.pallas import tpu_sc as plsc

_D = 512
_SCALE = math.sqrt(_D)
_NC, _NS = 2, 16
_NW = _NC * _NS
_CHUNK = 64
_NBUF = 3
_LANES = 16


def _make_scaled_gather(bsz, seq, d):
    n = bsz * seq
    per_w = n // _NW
    w_per_b = seq // per_w
    n_chunks = per_w // _CHUNK
    mesh = plsc.VectorSubcoreMesh(
        core_axis_name="c", subcore_axis_name="s",
        num_cores=_NC, num_subcores=_NS)

    @functools.partial(
        pl.kernel,
        out_type=jax.ShapeDtypeStruct((bsz, seq, d), jnp.float32),
        mesh=mesh,
        compiler_params=pltpu.CompilerParams(skip_device_barrier=True),
        scratch_types=[
            pltpu.VMEM((per_w,), jnp.int32),
            *[pltpu.VMEM((_CHUNK, d), jnp.float32) for _ in range(_NBUF)],
            *[pltpu.SemaphoreType.DMA for _ in range(2 * _NBUF)],
        ],
    )
    def emb(x_hbm, lut_hbm, out_hbm, idx_v, *rest):
        rows = rest[:_NBUF]
        in_sems = rest[_NBUF:2 * _NBUF]
        out_sems = rest[2 * _NBUF:]
        wid = lax.axis_index("s") * _NC + lax.axis_index("c")
        b_i = wid // w_per_b
        col = (wid % w_per_b) * per_w
        pltpu.sync_copy(x_hbm.at[b_i, pl.ds(col, per_w)], idx_v)

        def start_gather(c):
            b = c % _NBUF
            return pltpu.async_copy(
                lut_hbm.at[idx_v.at[pl.ds(c * _CHUNK, _CHUNK)]],
                rows[b], in_sems[b])

        gathers = {0: start_gather(0)}
        stores = {}
        for c in range(n_chunks):
            b = c % _NBUF
            nxt = c + 1
            if nxt < n_chunks:
                if nxt - _NBUF in stores:
                    stores.pop(nxt - _NBUF).wait()
                gathers[nxt] = start_gather(nxt)
            gathers.pop(c).wait()

            def row_body(r, acc, _b=b):
                for j in range(d // _LANES):
                    sl = (r, pl.ds(j * _LANES, _LANES))
                    rows[_b][sl] = rows[_b][sl] * _SCALE
                return acc
            lax.fori_loop(0, _CHUNK, row_body, 0)

            stores[c] = pltpu.async_copy(
                rows[b],
                out_hbm.at[b_i, pl.ds(col + c * _CHUNK, _CHUNK)],
                out_sems[b])
        for c in sorted(stores):
            stores.pop(c).wait()

    return emb


def kernel(x, lut):
    bsz, seq = x.shape
    return _make_scaled_gather(bsz, seq, lut.shape[1])(
        x.astype(jnp.int32), lut)
